# baseline (speedup 1.0000x reference)
import functools
import jax
import jax.numpy as jnp
from jax.experimental import pallas as pl
from jax.experimental.pallas import tpu as pltpu

N = 10000
E = 32000
B = 256
EMB = 768
IMG = 1024
HF = 512
NH = 4
NC = 5


def _elu(y):
    return jnp.where(y > 0, y, jnp.exp(y) - 1.0)


def _dense(x, w, b, act=False, bm=2000):
    M, K = x.shape
    NO = w.shape[1]
    b2 = b.reshape(1, NO)

    def body(x_ref, w_ref, b_ref, o_ref):
        y = jnp.dot(x_ref[...], w_ref[...],
                    preferred_element_type=jnp.float32) + b_ref[...]
        if act:
            y = _elu(y)
        o_ref[...] = y

    return pl.pallas_call(
        body,
        grid=(M // bm,),
        in_specs=[
            pl.BlockSpec((bm, K), lambda i: (i, 0)),
            pl.BlockSpec((K, NO), lambda i: (0, 0)),
            pl.BlockSpec((1, NO), lambda i: (0, 0)),
        ],
        out_specs=pl.BlockSpec((bm, NO), lambda i: (i, 0)),
        out_shape=jax.ShapeDtypeStruct((M, NO), jnp.float32),
    )(x, w, b2)


def _gat_project(h, W, al, ar, bm=1000):
    M, K = h.shape

    def body(h_ref, w_ref, al_ref, ar_ref, f_ref, el_ref, er_ref):
        y = jnp.dot(h_ref[...], w_ref[...],
                    preferred_element_type=jnp.float32)
        f_ref[...] = y
        f3 = y.reshape(bm, NH, HF)
        el_ref[...] = jnp.sum(f3 * al_ref[...][None], axis=-1)
        er_ref[...] = jnp.sum(f3 * ar_ref[...][None], axis=-1)

    return pl.pallas_call(
        body,
        grid=(M // bm,),
        in_specs=[
            pl.BlockSpec((bm, K), lambda i: (i, 0)),
            pl.BlockSpec((K, NH * HF), lambda i: (0, 0)),
            pl.BlockSpec((NH, HF), lambda i: (0, 0)),
            pl.BlockSpec((NH, HF), lambda i: (0, 0)),
        ],
        out_specs=[
            pl.BlockSpec((bm, NH * HF), lambda i: (i, 0)),
            pl.BlockSpec((bm, NH), lambda i: (i, 0)),
            pl.BlockSpec((bm, NH), lambda i: (i, 0)),
        ],
        out_shape=[
            jax.ShapeDtypeStruct((M, NH * HF), jnp.float32),
            jax.ShapeDtypeStruct((M, NH), jnp.float32),
            jax.ShapeDtypeStruct((M, NH), jnp.float32),
        ],
    )(h, W, al, ar)


def _bn_dense(x, g, be, w, b):
    M, K = x.shape
    NO = w.shape[1]

    def body(x_ref, g_ref, be_ref, w_ref, b_ref, o_ref):
        x = x_ref[...]
        m = jnp.mean(x, axis=0, keepdims=True)
        v = jnp.mean((x - m) * (x - m), axis=0, keepdims=True)
        xn = (x - m) / jnp.sqrt(v + 1e-5) * g_ref[...] + be_ref[...]
        y = jnp.dot(xn, w_ref[...], preferred_element_type=jnp.float32)
        o_ref[...] = _elu(y + b_ref[...])

    return pl.pallas_call(
        body,
        in_specs=[pl.BlockSpec((M, K), lambda: (0, 0)),
                  pl.BlockSpec((1, K), lambda: (0, 0)),
                  pl.BlockSpec((1, K), lambda: (0, 0)),
                  pl.BlockSpec((K, NO), lambda: (0, 0)),
                  pl.BlockSpec((1, NO), lambda: (0, 0))],
        out_specs=pl.BlockSpec((M, NO), lambda: (0, 0)),
        out_shape=jax.ShapeDtypeStruct((M, NO), jnp.float32),
    )(x, g.reshape(1, K), be.reshape(1, K), w, b.reshape(1, NO))


def _final(x, g, be, w, b):
    M, K = x.shape
    NO = w.shape[1]

    def body(x_ref, g_ref, be_ref, w_ref, b_ref, o_ref):
        x = x_ref[...]
        m = jnp.mean(x, axis=0, keepdims=True)
        v = jnp.mean((x - m) * (x - m), axis=0, keepdims=True)
        xn = (x - m) / jnp.sqrt(v + 1e-5) * g_ref[...] + be_ref[...]
        y = jnp.dot(xn, w_ref[...], preferred_element_type=jnp.float32)
        o_ref[...] = y + b_ref[...]

    return pl.pallas_call(
        body,
        in_specs=[pl.BlockSpec((M, K), lambda: (0, 0)),
                  pl.BlockSpec((1, K), lambda: (0, 0)),
                  pl.BlockSpec((1, K), lambda: (0, 0)),
                  pl.BlockSpec((K, NO), lambda: (0, 0)),
                  pl.BlockSpec((1, NO), lambda: (0, 0))],
        out_specs=pl.BlockSpec((M, NO), lambda: (0, 0)),
        out_shape=jax.ShapeDtypeStruct((M, NO), jnp.float32),
    )(x, g.reshape(1, K), be.reshape(1, K), w, b.reshape(1, NO))


def _mlp_pool(h2, Wfc, bfc, Wh, bh, graph_ids, bm=1000):
    def body(h_ref, wfc_ref, bfc_ref, wh_ref, bh_ref, gid_ref,
             sum_ref, cnt_ref):
        i = pl.program_id(0)
        h = jnp.dot(h_ref[...], wfc_ref[...],
                    preferred_element_type=jnp.float32) + bfc_ref[...]
        h = _elu(h)
        for k in range(8):
            h = jnp.dot(h, wh_ref[k], preferred_element_type=jnp.float32)
            h = _elu(h + bh_ref[k].reshape(1, HF))
        gid = gid_ref[...]
        onehot = (gid == jax.lax.broadcasted_iota(jnp.int32, (1, B), 1)
                  ).astype(jnp.float32)
        part = jnp.dot(onehot.T, h, preferred_element_type=jnp.float32)
        pcnt = jnp.sum(onehot, axis=0).reshape(B, 1)

        @pl.when(i == 0)
        def _init():
            sum_ref[...] = jnp.zeros_like(sum_ref)
            cnt_ref[...] = jnp.zeros_like(cnt_ref)

        sum_ref[...] += part
        cnt_ref[...] += pcnt * jnp.ones((1, 128), jnp.float32)

    sums, cnts = pl.pallas_call(
        body,
        grid=(N // bm,),
        in_specs=[
            pl.BlockSpec((bm, NH * HF), lambda i: (i, 0)),
            pl.BlockSpec((NH * HF, HF), lambda i: (0, 0)),
            pl.BlockSpec((1, HF), lambda i: (0, 0)),
            pl.BlockSpec((8, HF, HF), lambda i: (0, 0, 0)),
            pl.BlockSpec((8, HF), lambda i: (0, 0)),
            pl.BlockSpec((bm, 1), lambda i: (i, 0)),
        ],
        out_specs=[
            pl.BlockSpec((B, HF), lambda i: (0, 0)),
            pl.BlockSpec((B, 128), lambda i: (0, 0)),
        ],
        out_shape=[
            jax.ShapeDtypeStruct((B, HF), jnp.float32),
            jax.ShapeDtypeStruct((B, 128), jnp.float32),
        ],
    )(h2, Wfc, bfc.reshape(1, HF), Wh, bh, graph_ids.reshape(N, 1))
    return sums, cnts[:, :1]


def _edge_head_body(h, src_ref, dst_ref, feat_ref, el_ref, er_ref, o_ref):
    o_ref[...] = jnp.zeros_like(o_ref)

    def body(e, carry):
        d_prev, accv, accd = carry
        s = src_ref[e]
        d = dst_ref[e]
        elv = el_ref[pl.ds(s, 1), pl.ds(h, 1)]
        erv = er_ref[pl.ds(d, 1), pl.ds(h, 1)]
        ev = elv + erv
        ev = jnp.where(ev > 0, ev, 0.2 * ev)
        ee = jnp.exp(ev)
        row = feat_ref[pl.ds(s, 1), :]
        is_new = d != d_prev

        @pl.when(is_new)
        def _flush():
            o_ref[pl.ds(d_prev, 1), :] = accv / (accd + 1e-9)

        accv = jnp.where(is_new, 0.0, accv) + ee * row
        accd = jnp.where(is_new, 0.0, accd) + ee
        return d, accv, accd

    d0 = dst_ref[0]
    dl, accv, accd = jax.lax.fori_loop(
        0, E, body,
        (d0, jnp.zeros((1, HF), jnp.float32), jnp.zeros((1, 1), jnp.float32)))
    o_ref[pl.ds(dl, 1), :] = accv / (accd + 1e-9)


def _gat_edges(feat, el, er, src_s, dst_s):
    outs = []
    for h in range(NH):
        feat_h = jax.lax.slice(feat, (0, h * HF), (N, (h + 1) * HF))
        out_h = pl.pallas_call(
            functools.partial(_edge_head_body, h),
            in_specs=[
                pl.BlockSpec(memory_space=pltpu.SMEM),
                pl.BlockSpec(memory_space=pltpu.SMEM),
                pl.BlockSpec((N, HF), lambda: (0, 0)),
                pl.BlockSpec((N, NH), lambda: (0, 0)),
                pl.BlockSpec((N, NH), lambda: (0, 0)),
            ],
            out_specs=pl.BlockSpec((N, HF), lambda: (0, 0)),
            out_shape=jax.ShapeDtypeStruct((N, HF), jnp.float32),
        )(src_s, dst_s, feat_h, el, er)
        outs.append(out_h)
    return jnp.concatenate(outs, axis=1)


def kernel(node_feat, func_emb, img_embedding, func_text_embedding,
           edge_index, graph_ids, W1, al1, ar1, b1, W2, al2, ar2, b2,
           Wfc, bfc, Wfo, bfo, Wtx, btx, Wsw, bsw, Whf, bhf, Wh, bh,
           Wfin, bfin, g_text, be_text, g_swin, be_swin, g_hbn, be_hbn,
           g_fbn, be_fbn):
    order = jnp.argsort(edge_index[1])
    src = edge_index[0][order]
    dst = edge_index[1][order]

    x = _bn_dense(img_embedding, g_swin, be_swin, Wsw, bsw)
    ft = _bn_dense(func_text_embedding, g_text, be_text, Wtx, btx)

    feat1, el1, er1 = _gat_project(node_feat, W1, al1, ar1)
    h1 = _gat_edges(feat1, el1, er1, src, dst) + b1.reshape(1, NH * HF)

    feat2, el2, er2 = _gat_project(h1, W2, al2, ar2)
    h2 = _gat_edges(feat2, el2, er2, src, dst) + b2.reshape(1, NH * HF)

    sums, cnt = _mlp_pool(h2, Wfc, bfc, Wh, bh, graph_ids)
    h_feature = sums / jnp.maximum(cnt, 1.0)
    h_feature = _bn_dense(h_feature, g_hbn, be_hbn, Whf, bhf)

    all_feats = jnp.concatenate([x, h_feature, ft], axis=1)
    return _final(all_feats, g_fbn, be_fbn, Wfin, bfin)


# speedup vs baseline: 3.3173x; 3.3173x over previous
import functools
import jax
import jax.numpy as jnp
from jax.experimental import pallas as pl
from jax.experimental.pallas import tpu as pltpu

N = 10000
E = 32000
B = 256
EMB = 768
IMG = 1024
HF = 512
NH = 4
NC = 5


def _elu(y):
    return jnp.where(y > 0, y, jnp.exp(y) - 1.0)


def _dense(x, w, b, act=False, bm=2000):
    M, K = x.shape
    NO = w.shape[1]
    b2 = b.reshape(1, NO)

    def body(x_ref, w_ref, b_ref, o_ref):
        y = jnp.dot(x_ref[...], w_ref[...],
                    preferred_element_type=jnp.float32) + b_ref[...]
        if act:
            y = _elu(y)
        o_ref[...] = y

    return pl.pallas_call(
        body,
        grid=(M // bm,),
        in_specs=[
            pl.BlockSpec((bm, K), lambda i: (i, 0)),
            pl.BlockSpec((K, NO), lambda i: (0, 0)),
            pl.BlockSpec((1, NO), lambda i: (0, 0)),
        ],
        out_specs=pl.BlockSpec((bm, NO), lambda i: (i, 0)),
        out_shape=jax.ShapeDtypeStruct((M, NO), jnp.float32),
    )(x, w, b2)


def _gat_project(h, W, al, ar, bm=1000):
    M, K = h.shape

    def body(h_ref, w_ref, al_ref, ar_ref, f_ref, el_ref, er_ref):
        y = jnp.dot(h_ref[...], w_ref[...],
                    preferred_element_type=jnp.float32)
        f_ref[...] = y
        f3 = y.reshape(bm, NH, HF)
        el_ref[...] = jnp.sum(f3 * al_ref[...][None], axis=-1)
        er_ref[...] = jnp.sum(f3 * ar_ref[...][None], axis=-1)

    return pl.pallas_call(
        body,
        grid=(M // bm,),
        in_specs=[
            pl.BlockSpec((bm, K), lambda i: (i, 0)),
            pl.BlockSpec((K, NH * HF), lambda i: (0, 0)),
            pl.BlockSpec((NH, HF), lambda i: (0, 0)),
            pl.BlockSpec((NH, HF), lambda i: (0, 0)),
        ],
        out_specs=[
            pl.BlockSpec((bm, NH * HF), lambda i: (i, 0)),
            pl.BlockSpec((bm, NH), lambda i: (i, 0)),
            pl.BlockSpec((bm, NH), lambda i: (i, 0)),
        ],
        out_shape=[
            jax.ShapeDtypeStruct((M, NH * HF), jnp.float32),
            jax.ShapeDtypeStruct((M, NH), jnp.float32),
            jax.ShapeDtypeStruct((M, NH), jnp.float32),
        ],
    )(h, W, al, ar)


def _bn_dense(x, g, be, w, b):
    M, K = x.shape
    NO = w.shape[1]

    def body(x_ref, g_ref, be_ref, w_ref, b_ref, o_ref):
        x = x_ref[...]
        m = jnp.mean(x, axis=0, keepdims=True)
        v = jnp.mean((x - m) * (x - m), axis=0, keepdims=True)
        xn = (x - m) / jnp.sqrt(v + 1e-5) * g_ref[...] + be_ref[...]
        y = jnp.dot(xn, w_ref[...], preferred_element_type=jnp.float32)
        o_ref[...] = _elu(y + b_ref[...])

    return pl.pallas_call(
        body,
        in_specs=[pl.BlockSpec((M, K), lambda: (0, 0)),
                  pl.BlockSpec((1, K), lambda: (0, 0)),
                  pl.BlockSpec((1, K), lambda: (0, 0)),
                  pl.BlockSpec((K, NO), lambda: (0, 0)),
                  pl.BlockSpec((1, NO), lambda: (0, 0))],
        out_specs=pl.BlockSpec((M, NO), lambda: (0, 0)),
        out_shape=jax.ShapeDtypeStruct((M, NO), jnp.float32),
    )(x, g.reshape(1, K), be.reshape(1, K), w, b.reshape(1, NO))


def _final(x, g, be, w, b):
    M, K = x.shape
    NO = w.shape[1]

    def body(x_ref, g_ref, be_ref, w_ref, b_ref, o_ref):
        x = x_ref[...]
        m = jnp.mean(x, axis=0, keepdims=True)
        v = jnp.mean((x - m) * (x - m), axis=0, keepdims=True)
        xn = (x - m) / jnp.sqrt(v + 1e-5) * g_ref[...] + be_ref[...]
        y = jnp.dot(xn, w_ref[...], preferred_element_type=jnp.float32)
        o_ref[...] = y + b_ref[...]

    return pl.pallas_call(
        body,
        in_specs=[pl.BlockSpec((M, K), lambda: (0, 0)),
                  pl.BlockSpec((1, K), lambda: (0, 0)),
                  pl.BlockSpec((1, K), lambda: (0, 0)),
                  pl.BlockSpec((K, NO), lambda: (0, 0)),
                  pl.BlockSpec((1, NO), lambda: (0, 0))],
        out_specs=pl.BlockSpec((M, NO), lambda: (0, 0)),
        out_shape=jax.ShapeDtypeStruct((M, NO), jnp.float32),
    )(x, g.reshape(1, K), be.reshape(1, K), w, b.reshape(1, NO))


def _mlp_pool(h2, Wfc, bfc, Wh, bh, graph_ids, bm=1000):
    def body(h_ref, wfc_ref, bfc_ref, wh_ref, bh_ref, gid_ref,
             sum_ref, cnt_ref):
        i = pl.program_id(0)
        h = jnp.dot(h_ref[...], wfc_ref[...],
                    preferred_element_type=jnp.float32) + bfc_ref[...]
        h = _elu(h)
        for k in range(8):
            h = jnp.dot(h, wh_ref[k], preferred_element_type=jnp.float32)
            h = _elu(h + bh_ref[k].reshape(1, HF))
        gid = gid_ref[...]
        onehot = (gid == jax.lax.broadcasted_iota(jnp.int32, (1, B), 1)
                  ).astype(jnp.float32)
        part = jnp.dot(onehot.T, h, preferred_element_type=jnp.float32)
        pcnt = jnp.sum(onehot, axis=0).reshape(B, 1)

        @pl.when(i == 0)
        def _init():
            sum_ref[...] = jnp.zeros_like(sum_ref)
            cnt_ref[...] = jnp.zeros_like(cnt_ref)

        sum_ref[...] += part
        cnt_ref[...] += pcnt * jnp.ones((1, 128), jnp.float32)

    sums, cnts = pl.pallas_call(
        body,
        grid=(N // bm,),
        in_specs=[
            pl.BlockSpec((bm, NH * HF), lambda i: (i, 0)),
            pl.BlockSpec((NH * HF, HF), lambda i: (0, 0)),
            pl.BlockSpec((1, HF), lambda i: (0, 0)),
            pl.BlockSpec((8, HF, HF), lambda i: (0, 0, 0)),
            pl.BlockSpec((8, HF), lambda i: (0, 0)),
            pl.BlockSpec((bm, 1), lambda i: (i, 0)),
        ],
        out_specs=[
            pl.BlockSpec((B, HF), lambda i: (0, 0)),
            pl.BlockSpec((B, 128), lambda i: (0, 0)),
        ],
        out_shape=[
            jax.ShapeDtypeStruct((B, HF), jnp.float32),
            jax.ShapeDtypeStruct((B, 128), jnp.float32),
        ],
    )(h2, Wfc, bfc.reshape(1, HF), Wh, bh, graph_ids.reshape(N, 1))
    return sums, cnts[:, :1]


CH = 1000
EBA = 2000
EBC = 1280
NBD = 1000
NCH = N // CH


def _edge_logits(s2, d2, elb, erb):
    def body(s_ref, d_ref, el_ref, er_ref, o_ref):
        s = s_ref[...]
        d = d_ref[...]

        def step(nc, carry):
            elg, erg = carry
            col = jax.lax.broadcasted_iota(jnp.int32, (EBA, CH), 1) + nc * CH
            ohs = (s == col).astype(jnp.bfloat16)
            ohd = (d == col).astype(jnp.bfloat16)
            elg = elg + jnp.dot(ohs, el_ref[pl.ds(nc * CH, CH), :],
                                preferred_element_type=jnp.float32)
            erg = erg + jnp.dot(ohd, er_ref[pl.ds(nc * CH, CH), :],
                                preferred_element_type=jnp.float32)
            return elg, erg

        z = jnp.zeros((EBA, NH), jnp.float32)
        elg, erg = jax.lax.fori_loop(0, NCH, step, (z, z))
        e = elg + erg
        e = jnp.where(e > 0, e, 0.2 * e)
        o_ref[...] = jnp.exp(e)

    return pl.pallas_call(
        body,
        grid=(E // EBA,),
        in_specs=[
            pl.BlockSpec((EBA, 1), lambda i: (i, 0)),
            pl.BlockSpec((EBA, 1), lambda i: (i, 0)),
            pl.BlockSpec((N, NH), lambda i: (0, 0)),
            pl.BlockSpec((N, NH), lambda i: (0, 0)),
        ],
        out_specs=pl.BlockSpec((EBA, NH), lambda i: (i, 0)),
        out_shape=jax.ShapeDtypeStruct((E, NH), jnp.float32),
    )(s2, d2, elb, erb)


def _edge_den(d2r, ee):
    def body(d_ref, ee_ref, o_ref):
        i = pl.program_id(0)

        def step(ec, acc):
            drow = d_ref[:, pl.ds(ec * EBC, EBC)]
            rowi = jax.lax.broadcasted_iota(jnp.int32, (CH, EBC), 0) + i * CH
            oht = (rowi == drow).astype(jnp.bfloat16)
            eec = ee_ref[pl.ds(ec * EBC, EBC), :].astype(jnp.bfloat16)
            return acc + jnp.dot(oht, eec, preferred_element_type=jnp.float32)

        o_ref[...] = jax.lax.fori_loop(
            0, E // EBC, step, jnp.zeros((CH, NH), jnp.float32))

    return pl.pallas_call(
        body,
        grid=(NCH,),
        in_specs=[
            pl.BlockSpec((1, E), lambda i: (0, 0)),
            pl.BlockSpec((E, NH), lambda i: (0, 0)),
        ],
        out_specs=pl.BlockSpec((CH, NH), lambda i: (i, 0)),
        out_shape=jax.ShapeDtypeStruct((N, NH), jnp.float32),
    )(d2r, ee)


def _edge_msg(s2, d2, ee, featb, denb):
    def body(s_ref, d_ref, ee_ref, f_ref, den_ref, o_ref, acc_ref, dg_ref):
        nc = pl.program_id(1)

        @pl.when(nc == 0)
        def _init():
            acc_ref[...] = jnp.zeros_like(acc_ref)
            dg_ref[...] = jnp.zeros_like(dg_ref)

        col = jax.lax.broadcasted_iota(jnp.int32, (EBC, CH), 1) + nc * CH
        ohs = (s_ref[...] == col).astype(jnp.bfloat16)
        ohd = (d_ref[...] == col).astype(jnp.bfloat16)
        acc_ref[...] += jnp.dot(ohs, f_ref[...],
                                preferred_element_type=jnp.float32)
        dg_ref[...] += jnp.dot(ohd, den_ref[...],
                               preferred_element_type=jnp.float32)

        @pl.when(nc == NCH - 1)
        def _fin():
            alpha = ee_ref[...] / (dg_ref[...] + 1e-9)
            m = acc_ref[...].reshape(EBC, NH, HF) * alpha[:, :, None]
            o_ref[...] = m.reshape(EBC, NH * HF).astype(jnp.bfloat16)

    return pl.pallas_call(
        body,
        grid=(E // EBC, NCH),
        in_specs=[
            pl.BlockSpec((EBC, 1), lambda eb, nc: (eb, 0)),
            pl.BlockSpec((EBC, 1), lambda eb, nc: (eb, 0)),
            pl.BlockSpec((EBC, NH), lambda eb, nc: (eb, 0)),
            pl.BlockSpec((CH, NH * HF), lambda eb, nc: (nc, 0)),
            pl.BlockSpec((CH, NH), lambda eb, nc: (nc, 0)),
        ],
        out_specs=pl.BlockSpec((EBC, NH * HF), lambda eb, nc: (eb, 0)),
        out_shape=jax.ShapeDtypeStruct((E, NH * HF), jnp.bfloat16),
        scratch_shapes=[
            pltpu.VMEM((EBC, NH * HF), jnp.float32),
            pltpu.VMEM((EBC, NH), jnp.float32),
        ],
    )(s2, d2, ee, featb, denb)


def _edge_scatter(d2r, mg):
    def body(d_ref, m_ref, o_ref):
        nb = pl.program_id(0)
        eb = pl.program_id(1)

        @pl.when(eb == 0)
        def _init():
            o_ref[...] = jnp.zeros_like(o_ref)

        rowi = jax.lax.broadcasted_iota(jnp.int32, (NBD, EBC), 0) + nb * NBD
        oht = (rowi == d_ref[...]).astype(jnp.bfloat16)
        o_ref[...] += jnp.dot(oht, m_ref[...],
                              preferred_element_type=jnp.float32)

    return pl.pallas_call(
        body,
        grid=(N // NBD, E // EBC),
        in_specs=[
            pl.BlockSpec((1, EBC), lambda nb, eb: (0, eb)),
            pl.BlockSpec((EBC, NH * HF), lambda nb, eb: (eb, 0)),
        ],
        out_specs=pl.BlockSpec((NBD, NH * HF), lambda nb, eb: (nb, 0)),
        out_shape=jax.ShapeDtypeStruct((N, NH * HF), jnp.float32),
    )(d2r, mg)


def _gat_edges(feat, el, er, src, dst):
    s2 = src.reshape(E, 1)
    d2 = dst.reshape(E, 1)
    d2r = dst.reshape(1, E)
    elb = el.astype(jnp.bfloat16)
    erb = er.astype(jnp.bfloat16)
    featb = feat.astype(jnp.bfloat16)
    ee = _edge_logits(s2, d2, elb, erb)
    den = _edge_den(d2r, ee)
    mg = _edge_msg(s2, d2, ee, featb, den.astype(jnp.bfloat16))
    return _edge_scatter(d2r, mg)


def kernel(node_feat, func_emb, img_embedding, func_text_embedding,
           edge_index, graph_ids, W1, al1, ar1, b1, W2, al2, ar2, b2,
           Wfc, bfc, Wfo, bfo, Wtx, btx, Wsw, bsw, Whf, bhf, Wh, bh,
           Wfin, bfin, g_text, be_text, g_swin, be_swin, g_hbn, be_hbn,
           g_fbn, be_fbn):
    src = edge_index[0]
    dst = edge_index[1]

    x = _bn_dense(img_embedding, g_swin, be_swin, Wsw, bsw)
    ft = _bn_dense(func_text_embedding, g_text, be_text, Wtx, btx)

    feat1, el1, er1 = _gat_project(node_feat, W1, al1, ar1)
    h1 = _gat_edges(feat1, el1, er1, src, dst) + b1.reshape(1, NH * HF)

    feat2, el2, er2 = _gat_project(h1, W2, al2, ar2)
    h2 = _gat_edges(feat2, el2, er2, src, dst) + b2.reshape(1, NH * HF)

    sums, cnt = _mlp_pool(h2, Wfc, bfc, Wh, bh, graph_ids)
    h_feature = sums / jnp.maximum(cnt, 1.0)
    h_feature = _bn_dense(h_feature, g_hbn, be_hbn, Whf, bhf)

    all_feats = jnp.concatenate([x, h_feature, ft], axis=1)
    return _final(all_feats, g_fbn, be_fbn, Wfin, bfin)
